# baseline (device time: 21821 ns/iter reference)
import jax
import jax.numpy as jnp
from jax import lax
from jax.experimental import pallas as pl
from jax.experimental.pallas import tpu as pltpu

K = 8
R = 2


def kernel(x):
    m, n = x.shape
    mb = m // 4
    cr = mb // K
    h = cr // 2

    def body(x_ref, out_ref, xblk, raw, yrecv, summ, xrecv, zrecv, drecv,
             ssems, rsems, in_sem, out_sems):
        mx = lax.axis_index("x")
        my = lax.axis_index("y")
        mz = lax.axis_index("z")
        y_peer = (mx, 1 - my, mz)
        x_peer = (1 - mx, my, mz)
        z_peer = (mx, my, 1 - mz)
        d_peer = (1 - mx, my, 1 - mz)
        b_mine = 2 * mx + mz
        b_x = 2 * (1 - mx) + mz
        b_z = 2 * mx + (1 - mz)
        b_d = 2 * (1 - mx) + (1 - mz)

        in_dma = pltpu.make_async_copy(
            x_ref.at[pl.ds(b_mine * mb, mb), :], xblk, in_sem,
        )
        in_dma.start()

        barrier_sem = pltpu.get_barrier_semaphore()
        for p in (y_peer, x_peer, z_peer, d_peer):
            pl.semaphore_signal(
                barrier_sem, inc=1, device_id=p,
                device_id_type=pl.DeviceIdType.MESH,
            )
        in_dma.wait()

        def exchange(src, dst, i, c, peer, off=0, rows=cr):
            return pltpu.make_async_remote_copy(
                src_ref=src.at[pl.ds(c * cr + off, rows), :],
                dst_ref=dst.at[pl.ds(c * cr + off, rows), :],
                send_sem=ssems.at[i, c], recv_sem=rsems.at[i, c],
                device_id=peer, device_id_type=pl.DeviceIdType.MESH,
            )

        raw[...] = xblk[...].astype(jnp.bfloat16)
        pl.semaphore_wait(barrier_sem, 4)
        y_rdmas = []
        for c in range(K):
            r = exchange(raw, yrecv, 0, c, y_peer)
            r.start()
            y_rdmas.append(r)

        out_dmas = [None] * 4

        def flush(region, src, block_idx):
            d = pltpu.make_async_copy(
                src,
                out_ref.at[pl.ds(block_idx * mb, mb), :],
                out_sems.at[region],
            )
            d.start()
            out_dmas[region] = d

        x_rdmas, z_rdmas, d_rdmas = [], [], {}
        for c in range(K):
            cs = pl.ds(c * cr, cr)
            y_rdmas[c].wait_recv()
            summ[cs, :] = raw[cs, :] + yrecv[cs, :]
            for lst, dst, i, peer in (
                (x_rdmas, xrecv, 1, x_peer),
                (z_rdmas, zrecv, 2, z_peer),
            ):
                r = exchange(summ, dst, i, c, peer)
                r.start()
                lst.append(r)
            if c >= K - R:
                r = exchange(summ, drecv, 5, c, d_peer)
                r.start()
                d_rdmas[c] = r
        flush(0, summ, b_mine)

        fwd_a, fwd_b = [], []
        for c in range(K):
            x_rdmas[c].wait_recv()
            if c < K - R:
                fa = exchange(xrecv, drecv, 3, c, z_peer, off=0, rows=h)
                fa.start()
                fwd_a.append(fa)
            z_rdmas[c].wait_recv()
            if c < K - R:
                fb = exchange(zrecv, drecv, 4, c, x_peer, off=h, rows=h)
                fb.start()
                fwd_b.append(fb)
        flush(1, xrecv, b_x)
        flush(2, zrecv, b_z)

        for c in range(K - R):
            fwd_a[c].wait_recv()
            fwd_b[c].wait_recv()
        for c in d_rdmas:
            d_rdmas[c].wait_recv()
        flush(3, drecv, b_d)

        for c in range(K):
            y_rdmas[c].wait_send()
            x_rdmas[c].wait_send()
            z_rdmas[c].wait_send()
        for c in range(K - R):
            fwd_a[c].wait_send()
            fwd_b[c].wait_send()
        for c in d_rdmas:
            d_rdmas[c].wait_send()
        for d in out_dmas:
            d.wait()

    return pl.pallas_call(
        body,
        out_shape=jax.ShapeDtypeStruct((m, n), jnp.bfloat16),
        in_specs=[pl.BlockSpec(memory_space=pl.ANY)],
        out_specs=pl.BlockSpec(memory_space=pl.ANY),
        scratch_shapes=[
            pltpu.VMEM((mb, n), jnp.float32),
            pltpu.VMEM((mb, n), jnp.bfloat16),
            pltpu.VMEM((mb, n), jnp.bfloat16),
            pltpu.VMEM((mb, n), jnp.bfloat16),
            pltpu.VMEM((mb, n), jnp.bfloat16),
            pltpu.VMEM((mb, n), jnp.bfloat16),
            pltpu.VMEM((mb, n), jnp.bfloat16),
            pltpu.SemaphoreType.DMA((6, K)),
            pltpu.SemaphoreType.DMA((6, K)),
            pltpu.SemaphoreType.DMA,
            pltpu.SemaphoreType.DMA((4,)),
        ],
        compiler_params=pltpu.CompilerParams(collective_id=0),
    )(x)


# device time: 21606 ns/iter; 1.0100x vs baseline; 1.0100x over previous
import jax
import jax.numpy as jnp
from jax import lax
from jax.experimental import pallas as pl
from jax.experimental.pallas import tpu as pltpu

K = 16
R = 0


def kernel(x):
    m, n = x.shape
    mb = m // 4
    cr = mb // K
    h = cr // 2

    def body(x_ref, out_ref, xblk, raw, yrecv, summ, xrecv, zrecv, drecv,
             ssems, rsems, in_sem, out_sems):
        mx = lax.axis_index("x")
        my = lax.axis_index("y")
        mz = lax.axis_index("z")
        y_peer = (mx, 1 - my, mz)
        x_peer = (1 - mx, my, mz)
        z_peer = (mx, my, 1 - mz)
        d_peer = (1 - mx, my, 1 - mz)
        b_mine = 2 * mx + mz
        b_x = 2 * (1 - mx) + mz
        b_z = 2 * mx + (1 - mz)
        b_d = 2 * (1 - mx) + (1 - mz)

        in_dma = pltpu.make_async_copy(
            x_ref.at[pl.ds(b_mine * mb, mb), :], xblk, in_sem,
        )
        in_dma.start()

        barrier_sem = pltpu.get_barrier_semaphore()
        for p in (y_peer, x_peer, z_peer):
            pl.semaphore_signal(
                barrier_sem, inc=1, device_id=p,
                device_id_type=pl.DeviceIdType.MESH,
            )
        in_dma.wait()

        def exchange(src, dst, i, c, peer, off=0, rows=cr):
            return pltpu.make_async_remote_copy(
                src_ref=src.at[pl.ds(c * cr + off, rows), :],
                dst_ref=dst.at[pl.ds(c * cr + off, rows), :],
                send_sem=ssems.at[i, c], recv_sem=rsems.at[i, c],
                device_id=peer, device_id_type=pl.DeviceIdType.MESH,
            )

        raw[...] = xblk[...].astype(jnp.bfloat16)
        pl.semaphore_wait(barrier_sem, 3)
        y_rdmas = []
        for c in range(K):
            r = exchange(raw, yrecv, 0, c, y_peer)
            r.start()
            y_rdmas.append(r)

        out_dmas = [None] * 4

        def flush(region, src, block_idx):
            d = pltpu.make_async_copy(
                src,
                out_ref.at[pl.ds(block_idx * mb, mb), :],
                out_sems.at[region],
            )
            d.start()
            out_dmas[region] = d

        x_rdmas, z_rdmas, d_rdmas = [], [], {}
        for c in range(K):
            cs = pl.ds(c * cr, cr)
            y_rdmas[c].wait_recv()
            summ[cs, :] = raw[cs, :] + yrecv[cs, :]
            for lst, dst, i, peer in (
                (x_rdmas, xrecv, 1, x_peer),
                (z_rdmas, zrecv, 2, z_peer),
            ):
                r = exchange(summ, dst, i, c, peer)
                r.start()
                lst.append(r)
            if c >= K - R:
                r = exchange(summ, drecv, 5, c, d_peer)
                r.start()
                d_rdmas[c] = r
        flush(0, summ, b_mine)

        fwd_a, fwd_b = [], []
        for c in range(K):
            x_rdmas[c].wait_recv()
            if c < K - R:
                fa = exchange(xrecv, drecv, 3, c, z_peer, off=0, rows=h)
                fa.start()
                fwd_a.append(fa)
            z_rdmas[c].wait_recv()
            if c < K - R:
                fb = exchange(zrecv, drecv, 4, c, x_peer, off=h, rows=h)
                fb.start()
                fwd_b.append(fb)
        flush(1, xrecv, b_x)
        flush(2, zrecv, b_z)

        for c in range(K - R):
            fwd_a[c].wait_recv()
            fwd_b[c].wait_recv()
        for c in d_rdmas:
            d_rdmas[c].wait_recv()
        flush(3, drecv, b_d)

        for c in range(K):
            y_rdmas[c].wait_send()
            x_rdmas[c].wait_send()
            z_rdmas[c].wait_send()
        for c in range(K - R):
            fwd_a[c].wait_send()
            fwd_b[c].wait_send()
        for c in d_rdmas:
            d_rdmas[c].wait_send()
        for d in out_dmas:
            d.wait()

    return pl.pallas_call(
        body,
        out_shape=jax.ShapeDtypeStruct((m, n), jnp.bfloat16),
        in_specs=[pl.BlockSpec(memory_space=pl.ANY)],
        out_specs=pl.BlockSpec(memory_space=pl.ANY),
        scratch_shapes=[
            pltpu.VMEM((mb, n), jnp.float32),
            pltpu.VMEM((mb, n), jnp.bfloat16),
            pltpu.VMEM((mb, n), jnp.bfloat16),
            pltpu.VMEM((mb, n), jnp.bfloat16),
            pltpu.VMEM((mb, n), jnp.bfloat16),
            pltpu.VMEM((mb, n), jnp.bfloat16),
            pltpu.VMEM((mb, n), jnp.bfloat16),
            pltpu.SemaphoreType.DMA((6, K)),
            pltpu.SemaphoreType.DMA((6, K)),
            pltpu.SemaphoreType.DMA,
            pltpu.SemaphoreType.DMA((4,)),
        ],
        compiler_params=pltpu.CompilerParams(collective_id=0),
    )(x)


# device time: 20883 ns/iter; 1.0449x vs baseline; 1.0346x over previous
import jax
import jax.numpy as jnp
from jax import lax
from jax.experimental import pallas as pl
from jax.experimental.pallas import tpu as pltpu

K = 8


def kernel(x):
    m, n = x.shape
    mb = m // 4
    cr = mb // K
    h = cr // 2

    def body(x_ref, out_ref, xblk, raw, yrecv, summ, xrecv, zrecv, drecv,
             ssems, rsems, in_sem, out_sems):
        mx = lax.axis_index("x")
        my = lax.axis_index("y")
        mz = lax.axis_index("z")
        y_peer = (mx, 1 - my, mz)
        x_peer = (1 - mx, my, mz)
        z_peer = (mx, my, 1 - mz)
        b_mine = 2 * mx + mz
        b_x = 2 * (1 - mx) + mz
        b_z = 2 * mx + (1 - mz)
        b_d = 2 * (1 - mx) + (1 - mz)

        in_dma = pltpu.make_async_copy(
            x_ref.at[pl.ds(b_mine * mb, mb), :], xblk, in_sem,
        )
        in_dma.start()

        barrier_sem = pltpu.get_barrier_semaphore()
        for p in (y_peer, x_peer, z_peer):
            pl.semaphore_signal(
                barrier_sem, inc=1, device_id=p,
                device_id_type=pl.DeviceIdType.MESH,
            )
        in_dma.wait()

        def exchange(src, dst, i, c, peer, off=0, rows=cr):
            return pltpu.make_async_remote_copy(
                src_ref=src.at[pl.ds(c * cr + off, rows), :],
                dst_ref=dst.at[pl.ds(c * cr + off, rows), :],
                send_sem=ssems.at[i, c], recv_sem=rsems.at[i, c],
                device_id=peer, device_id_type=pl.DeviceIdType.MESH,
            )

        raw[...] = xblk[...].astype(jnp.bfloat16)
        pl.semaphore_wait(barrier_sem, 3)
        y_rdmas = []
        for c in range(K):
            r = exchange(raw, yrecv, 0, c, y_peer)
            r.start()
            y_rdmas.append(r)

        out_dmas = [None] * 4

        def flush(region, src, block_idx):
            d = pltpu.make_async_copy(
                src,
                out_ref.at[pl.ds(block_idx * mb, mb), :],
                out_sems.at[region],
            )
            d.start()
            out_dmas[region] = d

        x_rdmas, z_rdmas = [], []
        for c in range(K):
            cs = pl.ds(c * cr, cr)
            y_rdmas[c].wait_recv()
            summ[cs, :] = raw[cs, :] + yrecv[cs, :]
            for lst, dst, i, peer in (
                (x_rdmas, xrecv, 1, x_peer),
                (z_rdmas, zrecv, 2, z_peer),
            ):
                r = exchange(summ, dst, i, c, peer)
                r.start()
                lst.append(r)
        flush(0, summ, b_mine)

        fwd_a, fwd_b = [], []
        for c in range(K):
            x_rdmas[c].wait_recv()
            fa = exchange(xrecv, drecv, 3, c, z_peer, off=0, rows=h)
            fa.start()
            fwd_a.append(fa)
            z_rdmas[c].wait_recv()
            fb = exchange(zrecv, drecv, 4, c, x_peer, off=h, rows=h)
            fb.start()
            fwd_b.append(fb)
        flush(1, xrecv, b_x)
        flush(2, zrecv, b_z)

        for c in range(K):
            fwd_a[c].wait_recv()
            fwd_b[c].wait_recv()
        flush(3, drecv, b_d)

        for c in range(K):
            y_rdmas[c].wait_send()
            x_rdmas[c].wait_send()
            z_rdmas[c].wait_send()
        for c in range(K):
            fwd_a[c].wait_send()
            fwd_b[c].wait_send()
        for d in out_dmas:
            d.wait()

    return pl.pallas_call(
        body,
        out_shape=jax.ShapeDtypeStruct((m, n), jnp.bfloat16),
        in_specs=[pl.BlockSpec(memory_space=pl.ANY)],
        out_specs=pl.BlockSpec(memory_space=pl.ANY),
        scratch_shapes=[
            pltpu.VMEM((mb, n), jnp.float32),
            pltpu.VMEM((mb, n), jnp.bfloat16),
            pltpu.VMEM((mb, n), jnp.bfloat16),
            pltpu.VMEM((mb, n), jnp.bfloat16),
            pltpu.VMEM((mb, n), jnp.bfloat16),
            pltpu.VMEM((mb, n), jnp.bfloat16),
            pltpu.VMEM((mb, n), jnp.bfloat16),
            pltpu.SemaphoreType.DMA((5, K)),
            pltpu.SemaphoreType.DMA((5, K)),
            pltpu.SemaphoreType.DMA,
            pltpu.SemaphoreType.DMA((4,)),
        ],
        compiler_params=pltpu.CompilerParams(collective_id=0),
    )(x)
